# baseline (speedup 1.0000x reference)
import functools

import jax
import jax.numpy as jnp
from jax import lax
from jax.experimental import pallas as pl
from jax.experimental.pallas import tpu as pltpu
from jax.experimental.pallas import tpu_sc as plsc

_TMP = 0.1
_FMP = 0.1

_ROWS_PER_CHUNK = 8


def _sc_body(spec_hbm, tf_hbm, out_hbm, tf_v, buf, *, B, Fd, T, tlen, flen,
             n_workers, n_cores):
    wid = lax.axis_index("s") * n_cores + lax.axis_index("c")
    per_w = B // n_workers

    zeros16 = jnp.zeros((16,), jnp.float32)
    lane = lax.iota(jnp.int32, 16)
    n_chunks = Fd // _ROWS_PER_CHUNK

    def do_batch(i, _):
        b = wid * per_w + i
        pltpu.sync_copy(tf_hbm.at[b], tf_v)
        t0v = tf_v[0]
        f0v = tf_v[1]
        t0s = t0v[0]
        f0s = f0v[0]
        k0 = lax.shift_right_logical(t0s, 4)
        k1 = lax.shift_right_logical(t0s + tlen - 1, 4)

        def do_chunk(c, _):
            r0 = c * _ROWS_PER_CHUNK
            pltpu.sync_copy(spec_hbm.at[b, pl.ds(r0, _ROWS_PER_CHUNK), :], buf)

            def zero_mid(k, _):
                for r in range(_ROWS_PER_CHUNK):
                    buf[r, pl.ds(k * 16, 16)] = zeros16
                return 0

            lax.fori_loop(k0 + 1, k1, zero_mid, 0)

            for ks in (k0, k1):
                col = ks * 16 + lane
                m = (col >= t0v) & (col < t0v + tlen)
                for r in range(_ROWS_PER_CHUNK):
                    x = buf[r, pl.ds(ks * 16, 16)]
                    buf[r, pl.ds(ks * 16, 16)] = jnp.where(m, 0.0, x)

            for r in range(_ROWS_PER_CHUNK):
                gr = r0 + r

                @pl.when((gr >= f0s) & (gr < f0s + flen))
                def _():
                    def zrow(k, _):
                        buf[r, pl.ds(k * 16, 16)] = zeros16
                        return 0

                    lax.fori_loop(0, T // 16, zrow, 0)

            pltpu.sync_copy(buf, out_hbm.at[b, pl.ds(r0, _ROWS_PER_CHUNK), :])
            return 0

        lax.fori_loop(0, n_chunks, do_chunk, 0)
        return 0

    lax.fori_loop(0, per_w, do_batch, 0)


def kernel(spec):
    B, C, Fd, T = spec.shape
    tlen = int(T * _TMP)
    flen = int(Fd * _FMP)
    kt = jax.random.fold_in(jax.random.key(1), 0)
    t0 = jax.random.randint(kt, (B,), 0, max(1, T - tlen + 1))
    kf = jax.random.fold_in(jax.random.key(1), 1)
    f0 = jax.random.randint(kf, (B,), 0, max(1, Fd - flen + 1))
    tf = jnp.broadcast_to(
        jnp.stack([t0, f0], axis=1)[:, :, None], (B, 2, 16)
    ).astype(jnp.int32)

    x = spec.reshape(B, C * Fd, T)
    info = plsc.get_sparse_core_info()
    n_workers = info.num_cores * info.num_subcores
    mesh = plsc.VectorSubcoreMesh(core_axis_name="c", subcore_axis_name="s")
    body = functools.partial(
        _sc_body, B=B, Fd=C * Fd, T=T, tlen=tlen, flen=flen,
        n_workers=n_workers, n_cores=info.num_cores,
    )
    out = pl.kernel(
        body,
        mesh=mesh,
        out_type=jax.ShapeDtypeStruct(x.shape, x.dtype),
        scratch_types=[
            pltpu.VMEM((2, 16), jnp.int32),
            pltpu.VMEM((_ROWS_PER_CHUNK, T), jnp.float32),
        ],
    )(x, tf)
    return out.reshape(B, C, Fd, T)
